# baseline (speedup 1.0000x reference)
import jax
import jax.numpy as jnp
from jax import lax
from jax.experimental import pallas as pl
from jax.experimental.pallas import tpu as pltpu

B, N, SEQ, MAXSEQ, CIN, COUT, GROUPS, EPS = 2, 10000, 32, 32, 128, 128, 4, 1e-5
BN = B * N
RB = 400
NCH = BN // RB
RB2 = 2000
NCH2 = BN // RB2


def _tc_weighted_sum(xg, wts):
    def body(xg_ref, w_ref, o_ref):
        xb = xg_ref[...].reshape(RB, SEQ, CIN)
        acc = jnp.zeros((RB, CIN), jnp.float32)
        for s in range(SEQ):
            acc = acc + xb[:, s, :] * w_ref[:, s][:, None]
        o_ref[...] = acc

    return pl.pallas_call(
        body,
        grid=(NCH,),
        in_specs=[
            pl.BlockSpec((RB * SEQ, CIN), lambda i: (i, 0)),
            pl.BlockSpec((RB, SEQ), lambda i: (i, 0)),
        ],
        out_specs=pl.BlockSpec((RB, CIN), lambda i: (i, 0)),
        out_shape=jax.ShapeDtypeStruct((BN, CIN), jnp.float32),
    )(xg, wts)


def _tc_gate_h(x2, dwsum, wgt, bg2):
    def body(x_ref, d_ref, wg_ref, bg_ref, o_ref):
        xb = x_ref[...]
        gate = jnp.dot(xb, wg_ref[...],
                       preferred_element_type=jnp.float32) + bg_ref[...]
        o_ref[...] = d_ref[...] * gate + xb

    return pl.pallas_call(
        body,
        grid=(NCH,),
        in_specs=[
            pl.BlockSpec((RB, CIN), lambda i: (i, 0)),
            pl.BlockSpec((RB, CIN), lambda i: (i, 0)),
            pl.BlockSpec((CIN, CIN), lambda i: (0, 0)),
            pl.BlockSpec((1, CIN), lambda i: (0, 0)),
        ],
        out_specs=pl.BlockSpec((RB, CIN), lambda i: (i, 0)),
        out_shape=jax.ShapeDtypeStruct((BN, CIN), jnp.float32),
    )(x2, dwsum, wgt, bg2)


def _tc_spiralconv(hg, w3, bsc2):
    def body(hg_ref, w3_ref, b_ref, o_ref):
        s = pl.program_id(1)
        part = jnp.dot(hg_ref[...], w3_ref[0],
                       preferred_element_type=jnp.float32)

        @pl.when(s == 0)
        def _():
            o_ref[...] = part + b_ref[...]

        @pl.when(s != 0)
        def _():
            o_ref[...] += part

    return pl.pallas_call(
        body,
        grid=(NCH2, SEQ),
        in_specs=[
            pl.BlockSpec((RB2, CIN), lambda i, s: (s * NCH2 + i, 0)),
            pl.BlockSpec((1, CIN, COUT), lambda i, s: (s, 0, 0)),
            pl.BlockSpec((1, COUT), lambda i, s: (0, 0)),
        ],
        out_specs=pl.BlockSpec((RB2, COUT), lambda i, s: (i, 0)),
        out_shape=jax.ShapeDtypeStruct((BN, COUT), jnp.float32),
    )(hg, w3, bsc2)


def _tc_pool(rows, rwcol, rob1):
    def body(rob_ref, r_ref, rw_ref, o_ref):
        rb = r_ref[...].reshape(RB, SEQ, COUT)
        tot = jnp.zeros((RB, COUT), jnp.float32)
        for j in range(SEQ):
            tot = tot + rb[:, j, :]
        sv = jnp.dot(tot, rw_ref[...],
                     preferred_element_type=jnp.float32) * (1.0 / SEQ)
        sv = jnp.abs(sv + rob_ref[0])
        sv = jnp.minimum(sv * MAXSEQ, MAXSEQ - 1.0)
        acc = jnp.zeros((RB, COUT), jnp.float32)
        for j in range(SEQ):
            wj = jnp.clip(sv + 1.0 - j, 0.0, 1.0)
            acc = acc + rb[:, j, :] * wj
        o_ref[...] = acc

    return pl.pallas_call(
        body,
        grid=(NCH,),
        in_specs=[
            pl.BlockSpec(memory_space=pltpu.SMEM),
            pl.BlockSpec((RB * SEQ, COUT), lambda i: (i, 0)),
            pl.BlockSpec((COUT, 1), lambda i: (0, 0)),
        ],
        out_specs=pl.BlockSpec((RB, COUT), lambda i: (i, 0)),
        out_shape=jax.ShapeDtypeStruct((BN, COUT), jnp.float32),
    )(rob1, rows, rwcol)


def _tc_groupnorm_out(yv, y, gamma2, beta2, alpha1):
    CG = COUT // GROUPS

    def body(alpha_ref, yv_ref, y_ref, ga_ref, be_ref, o_ref):
        a = alpha_ref[0]
        for b in range(B):
            for g in range(GROUPS):
                blk = yv_ref[pl.ds(b * N, N), pl.ds(g * CG, CG)]
                m = jnp.mean(blk)
                var = jnp.mean((blk - m) * (blk - m))
                yn = (blk - m) * lax.rsqrt(var + EPS)
                yn = yn * ga_ref[0, pl.ds(g * CG, CG)] + \
                    be_ref[0, pl.ds(g * CG, CG)]
                o_ref[pl.ds(b * N, N), pl.ds(g * CG, CG)] = (
                    a * yn + y_ref[pl.ds(b * N, N), pl.ds(g * CG, CG)])

    return pl.pallas_call(
        body,
        in_specs=[
            pl.BlockSpec(memory_space=pltpu.SMEM),
            pl.BlockSpec(memory_space=pltpu.VMEM),
            pl.BlockSpec(memory_space=pltpu.VMEM),
            pl.BlockSpec(memory_space=pltpu.VMEM),
            pl.BlockSpec(memory_space=pltpu.VMEM),
        ],
        out_specs=pl.BlockSpec(memory_space=pltpu.VMEM),
        out_shape=jax.ShapeDtypeStruct((BN, COUT), jnp.float32),
    )(alpha1, yv, y, gamma2, beta2)


def kernel(x, Wg, bg, dw_weight, Wsc, bsc, ro_w, ro_b, gn_gamma, gn_beta,
           alpha, beta, indices, dynamic_indices):
    x2 = x.reshape(BN, CIN)
    boff = (jnp.arange(B, dtype=jnp.int32) * N)[:, None, None]
    fidx_a = (indices[None] + boff).reshape(BN * SEQ)
    fidx_sm = (indices.T[:, None, :] + boff.reshape(1, B, 1)
               ).reshape(SEQ * BN)
    fidx_d = (dynamic_indices[None] + boff).reshape(BN * SEQ)
    wts = jnp.broadcast_to((dw_weight * beta[0])[None], (B, N, SEQ)
                           ).reshape(BN, SEQ)
    w3 = Wsc.reshape(COUT, SEQ, CIN).transpose(1, 2, 0)

    xg = jnp.take(x2, fidx_a, axis=0)
    dwsum = _tc_weighted_sum(xg, wts)
    h = _tc_gate_h(x2, dwsum, Wg.T, bg.reshape(1, CIN))
    hg = jnp.take(h, fidx_sm, axis=0)
    y = _tc_spiralconv(hg, w3, bsc.reshape(1, COUT))
    rows = jnp.take(y, fidx_d, axis=0)
    yv = _tc_pool(rows, ro_w.reshape(COUT, 1), ro_b.reshape(1))
    out = _tc_groupnorm_out(yv, y, gn_gamma.reshape(1, COUT),
                            gn_beta.reshape(1, COUT), alpha)
    return out.reshape(B, N, COUT)
